# Initial kernel scaffold; baseline (speedup 1.0000x reference)
import jax
import jax.numpy as jnp
from jax.experimental import pallas as pl


def kernel(similarity_matrix, memory_bank_matrix, num_neighbors, temperature):
    raise NotImplementedError("write your pallas kernel here")


# speedup vs baseline: 182.3476x; 182.3476x over previous
import functools

import jax
import jax.numpy as jnp
from jax.experimental import pallas as pl
from jax.experimental.pallas import tpu as pltpu

_BIG = 9000000000000000.0


def _centrality_kernel(mb_ref, out_ref):
    s = jnp.sum(mb_ref[...], axis=-1) / mb_ref.shape[-1]
    out_ref[...] = s.reshape(1, 1, -1)


def _loss_kernel(sim_ref, cent_ref, temp_ref, k_ref, out_ref, *, rblk, n):
    i = pl.program_id(0)
    sim = sim_ref[...]
    cent = cent_ref[...]
    temp = temp_ref[0, 0]
    k = k_ref[0, 0]
    big = jnp.float32(_BIG)

    rows = i * rblk + jax.lax.broadcasted_iota(jnp.int32, (rblk, n), 0)
    cols = jax.lax.broadcasted_iota(jnp.int32, (rblk, n), 1)
    diag = rows == cols

    xi = jax.lax.bitcast_convert_type(sim, jnp.int32)
    key = jnp.where(xi < 0, xi ^ jnp.int32(0x7FFFFFFF), xi)
    intmin = jnp.int32(-(2**31))
    key = jnp.where(diag, intmin, key)

    def bs_body(_, carry):
        lo, hi = carry
        mid = (lo >> 1) + (hi >> 1) + (lo & hi & 1)
        cnt = jnp.sum((key > mid).astype(jnp.int32), axis=1, keepdims=True)
        ge = cnt >= k
        return jnp.where(ge, mid + 1, lo), jnp.where(ge, hi, mid)

    lo0 = jnp.full((rblk, 1), intmin, jnp.int32)
    hi0 = jnp.full((rblk, 1), 2**31 - 1, jnp.int32)
    _, tk = jax.lax.fori_loop(0, 32, bs_body, (lo0, hi0))

    gt = key > tk
    cnt_gt = jnp.sum(gt.astype(jnp.int32), axis=1, keepdims=True)
    m = k - cnt_gt
    eq = jnp.logical_and(key == tk, jnp.logical_not(diag))
    eqi = eq.astype(jnp.int32)

    def tie_body(_, carry):
        lo, hi = carry
        mid = (lo + hi) >> 1
        c = jnp.sum(jnp.where(cols <= mid, eqi, 0), axis=1, keepdims=True)
        ge = c >= m
        return jnp.where(ge, lo, mid + 1), jnp.where(ge, mid, hi)

    jlo0 = jnp.zeros((rblk, 1), jnp.int32)
    jhi0 = jnp.full((rblk, 1), n - 1, jnp.int32)
    _, jsel = jax.lax.fori_loop(0, 13, tie_body, (jlo0, jhi0))

    neighbor = jnp.logical_or(gt, jnp.logical_and(eq, cols <= jsel))
    extended = jnp.logical_or(neighbor, diag)
    comp = jnp.logical_not(extended)

    min_s = jnp.min(jnp.where(comp, sim, big), axis=1, keepdims=True)
    max_s = jnp.max(jnp.where(comp, sim, -big), axis=1, keepdims=True)
    min_c = jnp.min(jnp.where(comp, cent, big), axis=1, keepdims=True)
    max_c = jnp.max(jnp.where(comp, cent, -big), axis=1, keepdims=True)

    norm_s = (sim - min_s) / (max_s - min_s)
    norm_c = (cent - min_c) / (max_c - min_c)
    adj = jnp.where(neighbor, norm_s - norm_c, -big) * temp

    amax = jnp.max(adj, axis=1, keepdims=True)
    e = jnp.exp(adj - amax)
    w = e / jnp.sum(e, axis=1, keepdims=True)
    w = jnp.where(neighbor, w, 0.0)
    w = jnp.where(diag, 1.0, w)

    msim = jnp.where(extended, sim, -big)
    lmax = jnp.max(msim, axis=1, keepdims=True)
    lse = lmax + jnp.log(jnp.sum(jnp.exp(msim - lmax), axis=1, keepdims=True))
    lp = msim - lse

    numer = jnp.sum(w * lp, axis=1)
    denom = jnp.sum(w, axis=1)
    row_loss = -numer / denom

    @pl.when(i == 0)
    def _():
        out_ref[...] = jnp.zeros_like(out_ref)

    out_ref[...] += (jnp.sum(row_loss) / n).reshape(1, 1)


def kernel(similarity_matrix, memory_bank_matrix, num_neighbors, temperature):
    n = similarity_matrix.shape[0]

    cblk = 256
    cent = pl.pallas_call(
        _centrality_kernel,
        grid=(n // cblk,),
        in_specs=[pl.BlockSpec((cblk, n), lambda i: (i, 0))],
        out_specs=pl.BlockSpec((1, 1, cblk), lambda i: (i, 0, 0)),
        out_shape=jax.ShapeDtypeStruct((n // cblk, 1, cblk), jnp.float32),
    )(memory_bank_matrix)
    cent = cent.reshape(1, n)

    rblk = 128
    loss = pl.pallas_call(
        functools.partial(_loss_kernel, rblk=rblk, n=n),
        grid=(n // rblk,),
        in_specs=[
            pl.BlockSpec((rblk, n), lambda i: (i, 0)),
            pl.BlockSpec((1, n), lambda i: (0, 0)),
            pl.BlockSpec((1, 1), lambda i: (0, 0)),
            pl.BlockSpec((1, 1), lambda i: (0, 0)),
        ],
        out_specs=pl.BlockSpec((1, 1), lambda i: (0, 0)),
        out_shape=jax.ShapeDtypeStruct((1, 1), jnp.float32),
    )(
        similarity_matrix,
        cent,
        jnp.asarray(temperature, jnp.float32).reshape(1, 1),
        jnp.asarray(num_neighbors, jnp.int32).reshape(1, 1),
    )
    return loss[0, 0]
